# baseline (speedup 1.0000x reference)
import functools

import jax
import jax.numpy as jnp
from jax.experimental import pallas as pl
from jax.experimental.pallas import tpu as pltpu

_CONF_THRESH = 0.5
_IOU_THRESH = 0.3
_VAR0, _VAR1 = 0.1, 0.2
_MAX_DET = 100
_LANES = 128
_EXP_HALF = 0x3F000000


def _nms_kernel(n_valid, l0, l1, l2, l3, c1, p0, p1, p2, p3, triu, ones,
                out_ref, k_ref, x1_ref, y1_ref, x2_ref, y2_ref, ar_ref):
    rows = l0.shape[0]

    p2v = p2[...]
    p3v = p3[...]
    cx = p0[...] + l0[...] * _VAR0 * p2v
    cy = p1[...] + l1[...] * _VAR0 * p3v
    w = p2v * jnp.exp(l2[...] * _VAR1)
    h = p3v * jnp.exp(l3[...] * _VAR1)
    x1 = cx - w / 2.0
    y1 = cy - h / 2.0
    x2 = cx + w / 2.0
    y2 = cy + h / 2.0

    row_i = jax.lax.broadcasted_iota(jnp.int32, (rows, _LANES), 0)
    col_i = jax.lax.broadcasted_iota(jnp.int32, (rows, _LANES), 1)
    in_bounds = row_i * _LANES + col_i < n_valid

    probs = jax.nn.sigmoid(c1[...])
    score = jnp.where(probs >= _CONF_THRESH, probs, 0.0)
    score = jnp.where(in_bounds, score, 0.0)

    area = jnp.maximum(x2 - x1, 0.0) * jnp.maximum(y2 - y1, 0.0)

    rowkey = 255 - row_i
    sbits = jnp.minimum(
        jax.lax.bitcast_convert_type(score, jnp.int32) - _EXP_HALF,
        0x7FFFFF)
    key = jnp.where(score > 0.0, (sbits << 8) + rowkey, rowkey - 256)
    deadkey = rowkey - 512

    x1_ref[...] = x1
    y1_ref[...] = y1
    x2_ref[...] = x2
    y2_ref[...] = y2
    ar_ref[...] = area
    k_ref[...] = key

    g0 = jnp.max(jnp.max(key, axis=0, keepdims=True), axis=1, keepdims=True)

    tri = triu[...]
    onesm = ones[...]
    li = jax.lax.broadcasted_iota(jnp.int32, (1, _LANES), 1)

    def step(i, g):
        k = k_ref[...]
        maskv = k == g
        am = jnp.max(jnp.where(maskv, 1.0, 0.0), axis=0, keepdims=True)
        pf = jnp.dot(am, tri, preferred_element_type=jnp.float32)
        sel = jnp.logical_and(maskv, pf == 1.0)

        x1v = x1_ref[...]
        y1v = y1_ref[...]
        x2v = x2_ref[...]
        y2v = y2_ref[...]
        arv = ar_ref[...]

        zero = jnp.float32(0.0)
        e1 = jnp.sum(jnp.where(sel, x1v, zero), axis=0, keepdims=True)
        e2 = jnp.sum(jnp.where(sel, y1v, zero), axis=0, keepdims=True)
        e3 = jnp.sum(jnp.where(sel, x2v, zero), axis=0, keepdims=True)
        e4 = jnp.sum(jnp.where(sel, y2v, zero), axis=0, keepdims=True)
        e5 = jnp.sum(jnp.where(sel, arv, zero), axis=0, keepdims=True)
        hp = jax.lax.Precision.HIGHEST
        bx1 = jnp.dot(e1, onesm, preferred_element_type=jnp.float32, precision=hp)
        by1 = jnp.dot(e2, onesm, preferred_element_type=jnp.float32, precision=hp)
        bx2 = jnp.dot(e3, onesm, preferred_element_type=jnp.float32, precision=hp)
        by2 = jnp.dot(e4, onesm, preferred_element_type=jnp.float32, precision=hp)
        bar = jnp.dot(e5, onesm, preferred_element_type=jnp.float32, precision=hp)

        xx1 = jnp.maximum(bx1, x1v)
        yy1 = jnp.maximum(by1, y1v)
        xx2 = jnp.minimum(bx2, x2v)
        yy2 = jnp.minimum(by2, y2v)
        iw = jnp.maximum(xx2 - xx1, 0.0)
        ih = jnp.maximum(yy2 - yy1, 0.0)
        inter = iw * ih
        iou = inter / (bar + arv - inter + 1e-9)
        supp = jnp.logical_or(iou > _IOU_THRESH, sel)
        k_new = jnp.where(supp, deadkey, k)
        k_ref[...] = k_new

        valid = g >= 0
        mval = jnp.where(
            valid,
            jax.lax.bitcast_convert_type((g >> 8) + _EXP_HALF, jnp.float32),
            0.0)
        rowvec = (jnp.where(li == 0, bx1, zero)
                  + jnp.where(li == 1, by1, zero)
                  + jnp.where(li == 2, bx2, zero)
                  + jnp.where(li == 3, by2, zero)
                  + jnp.where(li == 4, mval, zero))
        out_ref[pl.ds(i, 1), :] = jnp.where(valid, rowvec, zero)

        return jnp.max(jnp.max(k_new, axis=0, keepdims=True),
                       axis=1, keepdims=True)

    jax.lax.fori_loop(0, _MAX_DET, step, g0)


@jax.jit
def kernel(loc, conf, priors):
    n = loc.shape[0]
    rows = (n + _LANES - 1) // _LANES
    rows = ((rows + 7) // 8) * 8
    n_pad = rows * _LANES

    def col(a, j, fill):
        c = a[:, j]
        c = jnp.concatenate([c, jnp.full((n_pad - n,), fill, c.dtype)])
        return c.reshape(rows, _LANES)

    triu = jnp.triu(jnp.ones((_LANES, _LANES), jnp.float32))
    onesm = jnp.ones((_LANES, _LANES), jnp.float32)

    args = (
        col(loc, 0, 0.0), col(loc, 1, 0.0), col(loc, 2, 0.0), col(loc, 3, 0.0),
        col(conf, 1, -1e9),
        col(priors, 0, 0.0), col(priors, 1, 0.0), col(priors, 2, 0.0), col(priors, 3, 0.0),
        triu, onesm,
    )

    scratch = [pltpu.VMEM((rows, _LANES), jnp.int32)] + \
              [pltpu.VMEM((rows, _LANES), jnp.float32)] * 5
    out = pl.pallas_call(
        functools.partial(_nms_kernel, n),
        out_shape=jax.ShapeDtypeStruct((_MAX_DET, _LANES), jnp.float32),
        scratch_shapes=scratch,
    )(*args)
    return out[:, :5]


# speedup vs baseline: 1.4528x; 1.4528x over previous
import functools

import jax
import jax.numpy as jnp
from jax.experimental import pallas as pl
from jax.experimental.pallas import tpu as pltpu

_CONF_THRESH = 0.5
_IOU_THRESH = 0.3
_VAR0, _VAR1 = 0.1, 0.2
_MAX_DET = 100
_LANES = 128
_EXP_HALF = 0x3F000000


def _nms_kernel(n_valid, l0, l1, l2, l3, c1, p0, p1, p2, p3,
                out_ref, k_ref, x1_ref, y1_ref, x2_ref, y2_ref, ar_ref):
    rows = l0.shape[0]

    p2v = p2[...]
    p3v = p3[...]
    cx = p0[...] + l0[...] * _VAR0 * p2v
    cy = p1[...] + l1[...] * _VAR0 * p3v
    w = p2v * jnp.exp(l2[...] * _VAR1)
    h = p3v * jnp.exp(l3[...] * _VAR1)
    x1 = cx - w / 2.0
    y1 = cy - h / 2.0
    x2 = cx + w / 2.0
    y2 = cy + h / 2.0

    row_i = jax.lax.broadcasted_iota(jnp.int32, (rows, _LANES), 0)
    col_i = jax.lax.broadcasted_iota(jnp.int32, (rows, _LANES), 1)
    in_bounds = col_i * rows + row_i < n_valid

    probs = jax.nn.sigmoid(c1[...])
    score = jnp.where(probs >= _CONF_THRESH, probs, 0.0)
    score = jnp.where(in_bounds, score, 0.0)

    area = jnp.maximum(x2 - x1, 0.0) * jnp.maximum(y2 - y1, 0.0)

    lanekey = 127 - col_i
    sbits = jnp.minimum(
        jax.lax.bitcast_convert_type(score, jnp.int32) - _EXP_HALF,
        0x7FFFFF)
    key = jnp.where(score > 0.0, (sbits << 7) + lanekey, lanekey - 128)
    deadkey = lanekey - 256

    x1_ref[...] = x1
    y1_ref[...] = y1
    x2_ref[...] = x2
    y2_ref[...] = y2
    ar_ref[...] = area
    k_ref[...] = key

    g0 = jnp.max(jnp.max(key, axis=0, keepdims=True))

    li = jax.lax.broadcasted_iota(jnp.int32, (1, _LANES), 1)
    big = jnp.int32(rows + 1)

    def step(i, g):
        k = k_ref[...]
        sel0 = k == g
        rminv = jnp.min(jnp.where(sel0, row_i, big), axis=0, keepdims=True)
        sel = jnp.logical_and(sel0, row_i == rminv)

        x1v = x1_ref[...]
        y1v = y1_ref[...]
        x2v = x2_ref[...]
        y2v = y2_ref[...]
        arv = ar_ref[...]

        zero = jnp.float32(0.0)
        bx1 = jnp.sum(jnp.sum(jnp.where(sel, x1v, zero), axis=0, keepdims=True))
        by1 = jnp.sum(jnp.sum(jnp.where(sel, y1v, zero), axis=0, keepdims=True))
        bx2 = jnp.sum(jnp.sum(jnp.where(sel, x2v, zero), axis=0, keepdims=True))
        by2 = jnp.sum(jnp.sum(jnp.where(sel, y2v, zero), axis=0, keepdims=True))
        bar = jnp.sum(jnp.sum(jnp.where(sel, arv, zero), axis=0, keepdims=True))

        xx1 = jnp.maximum(bx1, x1v)
        yy1 = jnp.maximum(by1, y1v)
        xx2 = jnp.minimum(bx2, x2v)
        yy2 = jnp.minimum(by2, y2v)
        iw = jnp.maximum(xx2 - xx1, 0.0)
        ih = jnp.maximum(yy2 - yy1, 0.0)
        inter = iw * ih
        iou = inter / (bar + arv - inter + 1e-9)
        supp = jnp.logical_or(iou > _IOU_THRESH, sel)
        k_new = jnp.where(supp, deadkey, k)
        k_ref[...] = k_new

        valid = g >= 0
        mval = jnp.where(
            valid,
            jax.lax.bitcast_convert_type((g >> 7) + _EXP_HALF, jnp.float32),
            0.0)
        rowvec = (jnp.where(li == 0, bx1, zero)
                  + jnp.where(li == 1, by1, zero)
                  + jnp.where(li == 2, bx2, zero)
                  + jnp.where(li == 3, by2, zero)
                  + jnp.where(li == 4, mval, zero))
        out_ref[pl.ds(i, 1), :] = jnp.where(valid, rowvec, zero)

        return jnp.max(jnp.max(k_new, axis=0, keepdims=True))

    jax.lax.fori_loop(0, _MAX_DET, step, g0)


@jax.jit
def kernel(loc, conf, priors):
    n = loc.shape[0]
    rows = (n + _LANES - 1) // _LANES
    rows = ((rows + 7) // 8) * 8
    n_pad = rows * _LANES

    def col(a, j, fill):
        c = a[:, j]
        c = jnp.concatenate([c, jnp.full((n_pad - n,), fill, c.dtype)])
        return c.reshape(_LANES, rows).T

    args = (
        col(loc, 0, 0.0), col(loc, 1, 0.0), col(loc, 2, 0.0), col(loc, 3, 0.0),
        col(conf, 1, -1e9),
        col(priors, 0, 0.0), col(priors, 1, 0.0), col(priors, 2, 0.0), col(priors, 3, 0.0),
    )

    scratch = [pltpu.VMEM((rows, _LANES), jnp.int32)] + \
              [pltpu.VMEM((rows, _LANES), jnp.float32)] * 5
    out = pl.pallas_call(
        functools.partial(_nms_kernel, n),
        out_shape=jax.ShapeDtypeStruct((_MAX_DET, _LANES), jnp.float32),
        scratch_shapes=scratch,
    )(*args)
    return out[:, :5]
